# baseline (speedup 1.0000x reference)
import functools

import jax
import jax.numpy as jnp
from jax import lax
from jax.experimental import pallas as pl
from jax.experimental.pallas import tpu as pltpu
from jax.experimental.pallas import tpu_sc as plsc

NUM_HIDDENS = 768
N_EMBED = 8192

TR = 1152
TCH = 512
UNROLL = N_EMBED // TCH
_DIFF_SCALE = 12.5 / (4608 * NUM_HIDDENS)


def _argmin_body(f_ref, e_ref, ind_ref, sum_ref, e2_ref):
    r = pl.program_id(0)

    f = f_ref[...]

    @pl.when(r == 0)
    def _():
        e = e_ref[...]
        ones = jnp.ones((1, e.shape[1]), jnp.float32)
        e2_ref[...] = lax.dot_general(ones, e * e, (((1,), (1,)), ((), ())),
                                      preferred_element_type=jnp.float32)

    sumf = jnp.sum(f * f, axis=1, keepdims=True)
    iota = lax.broadcasted_iota(jnp.int32, (TR, TCH), 1).astype(jnp.float32)

    run_val = None
    run_idx = None
    for j in range(UNROLL):
        e_j = e_ref[pl.ds(j * TCH, TCH), :]
        s = lax.dot_general(f, e_j, (((1,), (1,)), ((), ())),
                            preferred_element_type=jnp.float32)
        e2_j = e2_ref[0, pl.ds(j * TCH, TCH)][None, :]
        dist = (sumf - 2.0 * s) + e2_j
        minv = jnp.min(dist, axis=1, keepdims=True)
        cand = jnp.where(dist == minv, iota, jnp.float32(TCH))
        lidx = jnp.min(cand, axis=1, keepdims=True) + jnp.float32(j * TCH)
        if j == 0:
            run_val, run_idx = minv, lidx
        else:
            upd = minv < run_val
            run_val = jnp.where(upd, minv, run_val)
            run_idx = jnp.where(upd, lidx, run_idx)

    ind_ref[0, 0, :] = run_idx[:, 0].astype(jnp.int32)
    tile_sum = (jnp.sum(run_val) * _DIFF_SCALE)[None, None]
    sum_ref[...] = jnp.where(r == 0, tile_sum, sum_ref[...] + tile_sum)


def _make_sc_gather(n_rows, d):
    info = plsc.get_sparse_core_info()
    nw = info.num_cores * info.num_subcores
    assert n_rows % nw == 0
    b_per_w = n_rows // nw
    mesh = plsc.VectorSubcoreMesh(core_axis_name="c", subcore_axis_name="s")

    @functools.partial(
        pl.kernel,
        mesh=mesh,
        out_type=jax.ShapeDtypeStruct((n_rows, d), jnp.float32),
        scratch_types=[
            pltpu.VMEM((b_per_w,), jnp.int32),
            pltpu.VMEM((b_per_w, d), jnp.float32),
            pltpu.SemaphoreType.DMA,
        ],
    )
    def gather_k(table_hbm, idx_hbm, out_hbm, idx_v, rows_v, sem):
        wid = lax.axis_index("s") * info.num_cores + lax.axis_index("c")
        base = wid * b_per_w
        pltpu.sync_copy(idx_hbm.at[pl.ds(base, b_per_w)], idx_v)
        pltpu.async_copy(table_hbm.at[idx_v], rows_v, sem).wait()
        pltpu.sync_copy(rows_v, out_hbm.at[pl.ds(base, b_per_w)])

    return gather_k


def kernel(z, embed_weight):
    B, H, ch = z.shape
    n = B * H

    flatten = z.reshape(n, ch)

    ind3, dist_sum = pl.pallas_call(
        _argmin_body,
        grid=(n // TR,),
        in_specs=[
            pl.BlockSpec((TR, ch), lambda r: (r, 0)),
            pl.BlockSpec((N_EMBED, ch), lambda r: (0, 0)),
        ],
        out_specs=[
            pl.BlockSpec((1, 1, TR), lambda r: (r, 0, 0)),
            pl.BlockSpec((1, 1), lambda r: (0, 0)),
        ],
        out_shape=[
            jax.ShapeDtypeStruct((n // TR, 1, TR), jnp.int32),
            jax.ShapeDtypeStruct((1, 1), jnp.float32),
        ],
        scratch_shapes=[
            pltpu.VMEM((1, N_EMBED), jnp.float32),
        ],
    )(flatten, embed_weight)

    ind_flat = ind3.reshape(n)
    gather = _make_sc_gather(n, ch)
    z_q = gather(embed_weight, ind_flat).reshape(B, H, ch)

    diff = dist_sum[0, 0]
    ind = ind_flat.reshape(B, H)
    return z_q, diff, ind


# speedup vs baseline: 1.0202x; 1.0202x over previous
import functools

import jax
import jax.numpy as jnp
from jax import lax
from jax.experimental import pallas as pl
from jax.experimental.pallas import tpu as pltpu
from jax.experimental.pallas import tpu_sc as plsc

NUM_HIDDENS = 768
N_EMBED = 8192

TR = 768
TCH = 512
UNROLL = N_EMBED // TCH
_DIFF_SCALE = 12.5 / (4608 * NUM_HIDDENS)


def _argmin_body(f_ref, e_ref, ind_ref, sum_ref, e2_ref):
    r = pl.program_id(0)

    f = f_ref[...]

    @pl.when(r == 0)
    def _():
        e = e_ref[...]
        ones = jnp.ones((1, e.shape[1]), jnp.float32)
        e2_ref[...] = lax.dot_general(ones, e * e, (((1,), (1,)), ((), ())),
                                      preferred_element_type=jnp.float32)

    sumf = jnp.sum(f * f, axis=1, keepdims=True)
    iota = lax.broadcasted_iota(jnp.int32, (TR, TCH), 1).astype(jnp.float32)

    run_val = None
    run_idx = None
    for j in range(UNROLL):
        e_j = e_ref[pl.ds(j * TCH, TCH), :]
        s = lax.dot_general(f, e_j, (((1,), (1,)), ((), ())),
                            preferred_element_type=jnp.float32)
        e2_j = e2_ref[0, pl.ds(j * TCH, TCH)][None, :]
        dist = (sumf - 2.0 * s) + e2_j
        minv = jnp.min(dist, axis=1, keepdims=True)
        cand = jnp.where(dist == minv, iota, jnp.float32(TCH))
        lidx = jnp.min(cand, axis=1, keepdims=True) + jnp.float32(j * TCH)
        if j == 0:
            run_val, run_idx = minv, lidx
        else:
            upd = minv < run_val
            run_val = jnp.where(upd, minv, run_val)
            run_idx = jnp.where(upd, lidx, run_idx)

    ind_ref[0, 0, :] = run_idx[:, 0].astype(jnp.int32)
    tile_sum = (jnp.sum(run_val) * _DIFF_SCALE)[None, None]
    sum_ref[...] = jnp.where(r == 0, tile_sum, sum_ref[...] + tile_sum)


def _make_sc_gather(n_rows, d):
    info = plsc.get_sparse_core_info()
    nw = info.num_cores * info.num_subcores
    assert n_rows % nw == 0
    b_per_w = n_rows // nw
    mesh = plsc.VectorSubcoreMesh(core_axis_name="c", subcore_axis_name="s")

    @functools.partial(
        pl.kernel,
        mesh=mesh,
        out_type=jax.ShapeDtypeStruct((n_rows, d), jnp.float32),
        scratch_types=[
            pltpu.VMEM((b_per_w,), jnp.int32),
            pltpu.VMEM((b_per_w, d), jnp.float32),
            pltpu.SemaphoreType.DMA,
        ],
    )
    def gather_k(table_hbm, idx_hbm, out_hbm, idx_v, rows_v, sem):
        wid = lax.axis_index("s") * info.num_cores + lax.axis_index("c")
        base = wid * b_per_w
        pltpu.sync_copy(idx_hbm.at[pl.ds(base, b_per_w)], idx_v)
        pltpu.async_copy(table_hbm.at[idx_v], rows_v, sem).wait()
        pltpu.sync_copy(rows_v, out_hbm.at[pl.ds(base, b_per_w)])

    return gather_k


def kernel(z, embed_weight):
    B, H, ch = z.shape
    n = B * H

    flatten = z.reshape(n, ch)

    ind3, dist_sum = pl.pallas_call(
        _argmin_body,
        grid=(n // TR,),
        in_specs=[
            pl.BlockSpec((TR, ch), lambda r: (r, 0)),
            pl.BlockSpec((N_EMBED, ch), lambda r: (0, 0)),
        ],
        out_specs=[
            pl.BlockSpec((1, 1, TR), lambda r: (r, 0, 0)),
            pl.BlockSpec((1, 1), lambda r: (0, 0)),
        ],
        out_shape=[
            jax.ShapeDtypeStruct((n // TR, 1, TR), jnp.int32),
            jax.ShapeDtypeStruct((1, 1), jnp.float32),
        ],
        scratch_shapes=[
            pltpu.VMEM((1, N_EMBED), jnp.float32),
        ],
    )(flatten, embed_weight)

    ind_flat = ind3.reshape(n)
    gather = _make_sc_gather(n, ch)
    z_q = gather(embed_weight, ind_flat).reshape(B, H, ch)

    diff = dist_sum[0, 0]
    ind = ind_flat.reshape(B, H)
    return z_q, diff, ind
